# Initial kernel scaffold; baseline (speedup 1.0000x reference)
import jax
import jax.numpy as jnp
from jax.experimental import pallas as pl


def kernel(raw_feats, uids, id_map, fe_bn_g, fe_bn_b, fe_w1, fe_b1, fe_w2, fe_b2, ne_ln_g, ne_ln_b, ne_w1, ne_b1, ne_w2, ne_b2, wt_ln_g, wt_ln_b, wt_w1, wt_b1, wt_w2, wt_b2, vh_ln_g, vh_ln_b, vh_w1, vh_b1, vh_w2, vh_b2):
    raise NotImplementedError("write your pallas kernel here")


# speedup vs baseline: 1.5366x; 1.5366x over previous
import functools

import jax
import jax.numpy as jnp
from jax import lax
from jax.experimental import pallas as pl
from jax.experimental.pallas import tpu as pltpu
from jax.experimental.pallas import tpu_sc as plsc

N = 50000
D = 128
NW = 32
ROWS_PER_STREAM = 112
NP = 50176
B = 512
GRID = NP // B
BH = 1000


def _leaky(x):
    return jnp.where(x >= 0, x, 0.01 * x)


def _make_gather(total_rows, table_rows, interpret=False):
    per_w = total_rows // NW
    n_streams = per_w // ROWS_PER_STREAM
    assert per_w % ROWS_PER_STREAM == 0 and total_rows % NW == 0
    mesh = plsc.VectorSubcoreMesh(core_axis_name="c", subcore_axis_name="s",
                                  num_cores=2, num_subcores=16)

    @functools.partial(
        pl.kernel,
        out_type=jax.ShapeDtypeStruct((total_rows, D), jnp.float32),
        mesh=mesh,
        scratch_types=[
            pltpu.VMEM((ROWS_PER_STREAM,), jnp.int32),
            pltpu.VMEM((ROWS_PER_STREAM, D), jnp.float32),
            pltpu.SemaphoreType.DMA,
        ],
        interpret=interpret,
    )
    def gk(table_hbm, idx_hbm, out_hbm, idx_v, rows_v, sem):
        wid = lax.axis_index("s") * 2 + lax.axis_index("c")
        base = wid * per_w

        def body(s, carry):
            off = base + s * ROWS_PER_STREAM
            pltpu.sync_copy(idx_hbm.at[pl.ds(off, ROWS_PER_STREAM)], idx_v)
            pltpu.async_copy(table_hbm.at[idx_v], rows_v, sem).wait()
            pltpu.sync_copy(rows_v, out_hbm.at[pl.ds(off, ROWS_PER_STREAM), :])
            return carry

        lax.fori_loop(0, n_streams, body, 0)

    return gk


def _stats_body(xb, out):
    @pl.when(pl.program_id(0) == 0)
    def _():
        out[...] = jnp.zeros_like(out)

    x = xb[...]
    s = jnp.sum(x, axis=0, keepdims=True)
    s2 = jnp.sum(x * x, axis=0, keepdims=True)
    out[...] += jnp.concatenate([s, s2], axis=0)


def _stats_call(raw_feats, interpret=False):
    nb = 50
    bs = N // nb
    return pl.pallas_call(
        _stats_body,
        grid=(nb,),
        in_specs=[pl.BlockSpec((bs, D), lambda i: (i, 0))],
        out_specs=pl.BlockSpec((2, D), lambda i: (0, 0)),
        out_shape=jax.ShapeDtypeStruct((2, D), jnp.float32),
        interpret=interpret,
    )(raw_feats)


def _femb_body(xb, w1t, b1, w2t, b2, e0, out):
    h = _leaky(jnp.dot(xb[...], w1t[...], preferred_element_type=jnp.float32)
               + b1[...])
    y = _leaky(jnp.dot(h, w2t[...], preferred_element_type=jnp.float32)
               + b2[...])
    rows = (pl.program_id(0) * B
            + lax.broadcasted_iota(jnp.int32, (B, 1), 0))
    out[...] = jnp.where(rows < N, y, e0[...])


def _femb_call(raws_sorted, w1t, b1, w2t, b2, e0, interpret=False):
    full = lambda r, c: pl.BlockSpec((r, c), lambda i: (0, 0))
    return pl.pallas_call(
        _femb_body,
        grid=(GRID,),
        in_specs=[
            pl.BlockSpec((B, D), lambda i: (i, 0)),
            full(D, D), full(1, D), full(D, D), full(1, D), full(1, D),
        ],
        out_specs=pl.BlockSpec((B, D), lambda i: (i, 0)),
        out_shape=jax.ShapeDtypeStruct((NP, D), jnp.float32),
        interpret=interpret,
    )(raws_sorted, w1t, b1, w2t, b2, e0)


def _round_body(lb, rb, initb, tabb, lng, lnb, w1t, b1, w2t, b2, out):
    x = jnp.concatenate([lb[...], rb[...]], axis=0)
    mu = jnp.mean(x, axis=-1, keepdims=True)
    xc = x - mu
    var = jnp.mean(xc * xc, axis=-1, keepdims=True)
    xn = xc * lax.rsqrt(var + 1e-5) * lng[...] + lnb[...]
    h = jnp.dot(xn, w1t[...], preferred_element_type=jnp.float32) + b1[...]
    a = h[:, :D]
    g = h[:, D:]
    sw = a * (g * (1.0 / (1.0 + jnp.exp(-g))))
    y = jnp.dot(sw, w2t[...], preferred_element_type=jnp.float32) + b2[...]
    y = _leaky(y)
    feats = (y[:B] + y[B:] + initb[...]) * (1.0 / 3.0)
    rows = (pl.program_id(0) * B
            + lax.broadcasted_iota(jnp.int32, (B, 1), 0))
    feats = jnp.where(rows < N, feats, 0.0)
    out[...] = tabb[...] + feats


def _round_call(g2, init, table, lng, lnb, w1t, b1, w2t, b2, interpret=False):
    full = lambda r, c: pl.BlockSpec((r, c), lambda i: (0, 0))
    return pl.pallas_call(
        _round_body,
        grid=(GRID,),
        in_specs=[
            pl.BlockSpec((B, D), lambda i: (i, 0)),
            pl.BlockSpec((B, D), lambda i: (i + GRID, 0)),
            pl.BlockSpec((B, D), lambda i: (i, 0)),
            pl.BlockSpec((B, D), lambda i: (i, 0)),
            full(1, D), full(1, D), full(D, 2 * D), full(1, 2 * D),
            full(D, D), full(1, D),
        ],
        out_specs=pl.BlockSpec((B, D), lambda i: (i, 0)),
        out_shape=jax.ShapeDtypeStruct((NP, D), jnp.float32),
        interpret=interpret,
    )(g2, g2, init, table, lng, lnb, w1t, b1, w2t, b2)


def _head_one(xv, lng, lnb, w1t, b1, w2t):
    mu = jnp.mean(xv, axis=-1, keepdims=True)
    xc = xv - mu
    var = jnp.mean(xc * xc, axis=-1, keepdims=True)
    xn = xc * lax.rsqrt(var + 1e-5) * lng + lnb
    h = jnp.dot(xn, w1t, preferred_element_type=jnp.float32) + b1
    a = h[:, :D]
    g = h[:, D:]
    sw = a * (g * (1.0 / (1.0 + jnp.exp(-g))))
    return jnp.dot(sw, w2t, preferred_element_type=jnp.float32)


def _head_body(xg, wlng, wlnb, ww1t, wb1, ww2t, wb2,
               vlng, vlnb, vw1t, vb1, vw2t, vb2, xo, wo, vo):
    xv = xg[...]
    xo[...] = xv
    wo[...] = _head_one(xv, wlng[...], wlnb[...], ww1t[...], wb1[...],
                        ww2t[...]) + wb2[...]
    vo[...] = jnp.tanh(
        _head_one(xv, vlng[...], vlnb[...], vw1t[...], vb1[...], vw2t[...])
        + vb2[...])


def _head_call(xg, wlng, wlnb, ww1t, wb1, ww2t, wb2,
               vlng, vlnb, vw1t, vb1, vw2t, vb2, interpret=False):
    full = lambda r, c: pl.BlockSpec((r, c), lambda i: (0, 0))
    nb = N // BH
    return pl.pallas_call(
        _head_body,
        grid=(nb,),
        in_specs=[
            pl.BlockSpec((BH, D), lambda i: (i, 0)),
            full(1, D), full(1, D), full(D, 2 * D), full(1, 2 * D),
            full(D, 1), full(1, 1),
            full(1, D), full(1, D), full(D, 2 * D), full(1, 2 * D),
            full(D, 1), full(1, 1),
        ],
        out_specs=[
            pl.BlockSpec((BH, D), lambda i: (i, 0)),
            pl.BlockSpec((BH, 1), lambda i: (i, 0)),
            pl.BlockSpec((BH, 1), lambda i: (i, 0)),
        ],
        out_shape=[
            jax.ShapeDtypeStruct((N, D), jnp.float32),
            jax.ShapeDtypeStruct((N, 1), jnp.float32),
            jax.ShapeDtypeStruct((N, 1), jnp.float32),
        ],
        interpret=interpret,
    )(xg, wlng, wlnb, ww1t, wb1, ww2t, wb2,
      vlng, vlnb, vw1t, vb1, vw2t, vb2)


def _run(raw_feats, uids, id_map, fe_bn_g, fe_bn_b, fe_w1, fe_b1, fe_w2,
         fe_b2, ne_ln_g, ne_ln_b, ne_w1, ne_b1, ne_w2, ne_b2, wt_ln_g,
         wt_ln_b, wt_w1, wt_b1, wt_w2, wt_b2, vh_ln_g, vh_ln_b, vh_w1,
         vh_b1, vh_w2, vh_b2, tc_interpret=False, sc_interpret=False):
    order = jnp.argsort(uids)
    zpad = jnp.zeros((NP - N,), jnp.int32)
    order_ext = jnp.concatenate([order.astype(jnp.int32), zpad])
    uids_ext = jnp.concatenate([uids.astype(jnp.int32), zpad])
    idx_l = jnp.concatenate([id_map[:, 0].astype(jnp.int32), zpad])
    idx_r = jnp.concatenate([id_map[:, 1].astype(jnp.int32), zpad])
    cidx = jnp.concatenate([idx_l, idx_r])

    sums = _stats_call(raw_feats, interpret=tc_interpret)
    m = sums[0] / (N + 1)
    var = sums[1] / (N + 1) - m * m
    s = fe_bn_g * lax.rsqrt(var + 1e-5)
    w1f = fe_w1 * s[None, :]
    b1f = fe_b1 + (fe_bn_b - m * s) @ fe_w1.T
    e0 = _leaky(_leaky(b1f) @ fe_w2.T + fe_b2)[None]

    gather_np = _make_gather(NP, N, interpret=sc_interpret)
    gather_2np = _make_gather(2 * NP, NP, interpret=sc_interpret)
    gather_fin = _make_gather(NP, NP, interpret=sc_interpret)

    raws_sorted = gather_np(raw_feats, order_ext)
    table = _femb_call(raws_sorted, w1f.T, b1f[None], fe_w2.T, fe_b2[None],
                       e0, interpret=tc_interpret)
    init = table

    rargs = (ne_ln_g[None], ne_ln_b[None], ne_w1.T, ne_b1[None], ne_w2.T,
             ne_b2[None])
    for _ in range(6):
        g2 = gather_2np(table, cidx)
        table = _round_call(g2, init, table, *rargs,
                            interpret=tc_interpret)

    xg = gather_fin(table, uids_ext)
    return _head_call(
        xg, wt_ln_g[None], wt_ln_b[None], wt_w1.T, wt_b1[None], wt_w2.T,
        wt_b2.reshape(1, 1),
        vh_ln_g[None], vh_ln_b[None], vh_w1.T, vh_b1[None], vh_w2.T,
        vh_b2.reshape(1, 1),
        interpret=tc_interpret)


def kernel(raw_feats, uids, id_map, fe_bn_g, fe_bn_b, fe_w1, fe_b1, fe_w2,
           fe_b2, ne_ln_g, ne_ln_b, ne_w1, ne_b1, ne_w2, ne_b2, wt_ln_g,
           wt_ln_b, wt_w1, wt_b1, wt_w2, wt_b2, vh_ln_g, vh_ln_b, vh_w1,
           vh_b1, vh_w2, vh_b2):
    return _run(raw_feats, uids, id_map, fe_bn_g, fe_bn_b, fe_w1, fe_b1,
                fe_w2, fe_b2, ne_ln_g, ne_ln_b, ne_w1, ne_b1, ne_w2, ne_b2,
                wt_ln_g, wt_ln_b, wt_w1, wt_b1, wt_w2, wt_b2, vh_ln_g,
                vh_ln_b, vh_w1, vh_b1, vh_w2, vh_b2)


# speedup vs baseline: 1.7563x; 1.1430x over previous
import functools

import jax
import jax.numpy as jnp
from jax import lax
from jax.experimental import pallas as pl
from jax.experimental.pallas import tpu as pltpu
from jax.experimental.pallas import tpu_sc as plsc

N = 50000
D = 128
NW = 32
ROWS_PER_STREAM = 112
NP = 50176
B = 512
GRID = NP // B
BH = 1000


def _leaky(x):
    return jnp.where(x >= 0, x, 0.01 * x)


def _make_gather(total_rows, table_rows, interpret=False):
    per_w = total_rows // NW
    ns = per_w // ROWS_PER_STREAM
    assert per_w % ROWS_PER_STREAM == 0 and total_rows % NW == 0
    assert ns % 2 == 0 and ns >= 4
    mesh = plsc.VectorSubcoreMesh(core_axis_name="c", subcore_axis_name="s",
                                  num_cores=2, num_subcores=16)
    R = ROWS_PER_STREAM

    @functools.partial(
        pl.kernel,
        out_type=jax.ShapeDtypeStruct((total_rows, D), jnp.float32),
        mesh=mesh,
        scratch_types=[
            pltpu.VMEM((per_w,), jnp.int32),
            pltpu.VMEM((R, D), jnp.float32),
            pltpu.VMEM((R, D), jnp.float32),
            pltpu.SemaphoreType.DMA,
            pltpu.SemaphoreType.DMA,
            pltpu.SemaphoreType.DMA,
            pltpu.SemaphoreType.DMA,
        ],
        interpret=interpret,
    )
    def gk(table_hbm, idx_hbm, out_hbm, idx_v, row0, row1, g0, g1, w0, w1):
        wid = lax.axis_index("s") * 2 + lax.axis_index("c")
        base = wid * per_w
        rows = (row0, row1)
        gsem = (g0, g1)
        wsem = (w0, w1)

        pltpu.sync_copy(idx_hbm.at[pl.ds(base, per_w)], idx_v)

        def gather(s, b):
            pltpu.async_copy(table_hbm.at[idx_v.at[pl.ds(s * R, R)]],
                             rows[b], gsem[b])

        def gwait(b):
            pltpu.make_async_copy(table_hbm.at[idx_v.at[pl.ds(0, R)]],
                                  rows[b], gsem[b]).wait()

        def wback(s, b):
            pltpu.async_copy(rows[b],
                             out_hbm.at[pl.ds(base + s * R, R), :], wsem[b])

        def wwait(b):
            pltpu.make_async_copy(rows[b],
                                  out_hbm.at[pl.ds(base, R), :], wsem[b]).wait()

        gather(0, 0)
        gather(1, 1)
        gwait(0)
        wback(0, 0)

        def body(i, carry):
            s2 = 2 + 2 * i
            for b in range(2):
                s = s2 + b
                wwait(b)
                gather(s, b)
                gwait(1 - b)
                wback(s - 1, 1 - b)
            return carry

        lax.fori_loop(0, (ns - 2) // 2, body, 0, unroll=False)

        gwait((ns - 1) % 2)
        wback(ns - 1, (ns - 1) % 2)
        wwait(0)
        wwait(1)

    return gk


def _stats_body(xb, out):
    @pl.when(pl.program_id(0) == 0)
    def _():
        out[...] = jnp.zeros_like(out)

    x = xb[...]
    s = jnp.sum(x, axis=0, keepdims=True)
    s2 = jnp.sum(x * x, axis=0, keepdims=True)
    out[...] += jnp.concatenate([s, s2], axis=0)


def _stats_call(raw_feats, interpret=False):
    nb = 50
    bs = N // nb
    return pl.pallas_call(
        _stats_body,
        grid=(nb,),
        in_specs=[pl.BlockSpec((bs, D), lambda i: (i, 0))],
        out_specs=pl.BlockSpec((2, D), lambda i: (0, 0)),
        out_shape=jax.ShapeDtypeStruct((2, D), jnp.float32),
        interpret=interpret,
    )(raw_feats)


def _femb_body(xb, w1t, b1, w2t, b2, e0, out):
    h = _leaky(jnp.dot(xb[...], w1t[...], preferred_element_type=jnp.float32)
               + b1[...])
    y = _leaky(jnp.dot(h, w2t[...], preferred_element_type=jnp.float32)
               + b2[...])
    rows = (pl.program_id(0) * B
            + lax.broadcasted_iota(jnp.int32, (B, 1), 0))
    out[...] = jnp.where(rows < N, y, e0[...])


def _femb_call(raws_sorted, w1t, b1, w2t, b2, e0, interpret=False):
    full = lambda r, c: pl.BlockSpec((r, c), lambda i: (0, 0))
    return pl.pallas_call(
        _femb_body,
        grid=(GRID,),
        in_specs=[
            pl.BlockSpec((B, D), lambda i: (i, 0)),
            full(D, D), full(1, D), full(D, D), full(1, D), full(1, D),
        ],
        out_specs=pl.BlockSpec((B, D), lambda i: (i, 0)),
        out_shape=jax.ShapeDtypeStruct((NP, D), jnp.float32),
        interpret=interpret,
    )(raws_sorted, w1t, b1, w2t, b2, e0)


def _round_body(lb, rb, initb, tabb, lng, lnb, w1t, b1, w2t, b2, out):
    x = jnp.concatenate([lb[...], rb[...]], axis=0)
    mu = jnp.mean(x, axis=-1, keepdims=True)
    xc = x - mu
    var = jnp.mean(xc * xc, axis=-1, keepdims=True)
    xn = xc * lax.rsqrt(var + 1e-5) * lng[...] + lnb[...]
    h = jnp.dot(xn, w1t[...], preferred_element_type=jnp.float32) + b1[...]
    a = h[:, :D]
    g = h[:, D:]
    sw = a * (g * (1.0 / (1.0 + jnp.exp(-g))))
    y = jnp.dot(sw, w2t[...], preferred_element_type=jnp.float32) + b2[...]
    y = _leaky(y)
    feats = (y[:B] + y[B:] + initb[...]) * (1.0 / 3.0)
    rows = (pl.program_id(0) * B
            + lax.broadcasted_iota(jnp.int32, (B, 1), 0))
    feats = jnp.where(rows < N, feats, 0.0)
    out[...] = tabb[...] + feats


def _round_call(g2, init, table, lng, lnb, w1t, b1, w2t, b2, interpret=False):
    full = lambda r, c: pl.BlockSpec((r, c), lambda i: (0, 0))
    return pl.pallas_call(
        _round_body,
        grid=(GRID,),
        in_specs=[
            pl.BlockSpec((B, D), lambda i: (i, 0)),
            pl.BlockSpec((B, D), lambda i: (i + GRID, 0)),
            pl.BlockSpec((B, D), lambda i: (i, 0)),
            pl.BlockSpec((B, D), lambda i: (i, 0)),
            full(1, D), full(1, D), full(D, 2 * D), full(1, 2 * D),
            full(D, D), full(1, D),
        ],
        out_specs=pl.BlockSpec((B, D), lambda i: (i, 0)),
        out_shape=jax.ShapeDtypeStruct((NP, D), jnp.float32),
        interpret=interpret,
    )(g2, g2, init, table, lng, lnb, w1t, b1, w2t, b2)


def _head_one(xv, lng, lnb, w1t, b1, w2t):
    mu = jnp.mean(xv, axis=-1, keepdims=True)
    xc = xv - mu
    var = jnp.mean(xc * xc, axis=-1, keepdims=True)
    xn = xc * lax.rsqrt(var + 1e-5) * lng + lnb
    h = jnp.dot(xn, w1t, preferred_element_type=jnp.float32) + b1
    a = h[:, :D]
    g = h[:, D:]
    sw = a * (g * (1.0 / (1.0 + jnp.exp(-g))))
    return jnp.dot(sw, w2t, preferred_element_type=jnp.float32)


def _head_body(xg, wlng, wlnb, ww1t, wb1, ww2t, wb2,
               vlng, vlnb, vw1t, vb1, vw2t, vb2, xo, wo, vo):
    xv = xg[...]
    xo[...] = xv
    wo[...] = _head_one(xv, wlng[...], wlnb[...], ww1t[...], wb1[...],
                        ww2t[...]) + wb2[...]
    vo[...] = jnp.tanh(
        _head_one(xv, vlng[...], vlnb[...], vw1t[...], vb1[...], vw2t[...])
        + vb2[...])


def _head_call(xg, wlng, wlnb, ww1t, wb1, ww2t, wb2,
               vlng, vlnb, vw1t, vb1, vw2t, vb2, interpret=False):
    full = lambda r, c: pl.BlockSpec((r, c), lambda i: (0, 0))
    nb = N // BH
    return pl.pallas_call(
        _head_body,
        grid=(nb,),
        in_specs=[
            pl.BlockSpec((BH, D), lambda i: (i, 0)),
            full(1, D), full(1, D), full(D, 2 * D), full(1, 2 * D),
            full(D, 1), full(1, 1),
            full(1, D), full(1, D), full(D, 2 * D), full(1, 2 * D),
            full(D, 1), full(1, 1),
        ],
        out_specs=[
            pl.BlockSpec((BH, D), lambda i: (i, 0)),
            pl.BlockSpec((BH, 1), lambda i: (i, 0)),
            pl.BlockSpec((BH, 1), lambda i: (i, 0)),
        ],
        out_shape=[
            jax.ShapeDtypeStruct((N, D), jnp.float32),
            jax.ShapeDtypeStruct((N, 1), jnp.float32),
            jax.ShapeDtypeStruct((N, 1), jnp.float32),
        ],
        interpret=interpret,
    )(xg, wlng, wlnb, ww1t, wb1, ww2t, wb2,
      vlng, vlnb, vw1t, vb1, vw2t, vb2)


def _run(raw_feats, uids, id_map, fe_bn_g, fe_bn_b, fe_w1, fe_b1, fe_w2,
         fe_b2, ne_ln_g, ne_ln_b, ne_w1, ne_b1, ne_w2, ne_b2, wt_ln_g,
         wt_ln_b, wt_w1, wt_b1, wt_w2, wt_b2, vh_ln_g, vh_ln_b, vh_w1,
         vh_b1, vh_w2, vh_b2, tc_interpret=False, sc_interpret=False):
    order = jnp.argsort(uids)
    zpad = jnp.zeros((NP - N,), jnp.int32)
    order_ext = jnp.concatenate([order.astype(jnp.int32), zpad])
    uids_ext = jnp.concatenate([uids.astype(jnp.int32), zpad])
    idx_l = jnp.concatenate([id_map[:, 0].astype(jnp.int32), zpad])
    idx_r = jnp.concatenate([id_map[:, 1].astype(jnp.int32), zpad])
    cidx = jnp.concatenate([idx_l, idx_r])

    sums = _stats_call(raw_feats, interpret=tc_interpret)
    m = sums[0] / (N + 1)
    var = sums[1] / (N + 1) - m * m
    s = fe_bn_g * lax.rsqrt(var + 1e-5)
    w1f = fe_w1 * s[None, :]
    b1f = fe_b1 + (fe_bn_b - m * s) @ fe_w1.T
    e0 = _leaky(_leaky(b1f) @ fe_w2.T + fe_b2)[None]

    gather_np = _make_gather(NP, N, interpret=sc_interpret)
    gather_2np = _make_gather(2 * NP, NP, interpret=sc_interpret)
    gather_fin = _make_gather(NP, NP, interpret=sc_interpret)

    raws_sorted = gather_np(raw_feats, order_ext)
    table = _femb_call(raws_sorted, w1f.T, b1f[None], fe_w2.T, fe_b2[None],
                       e0, interpret=tc_interpret)
    init = table

    rargs = (ne_ln_g[None], ne_ln_b[None], ne_w1.T, ne_b1[None], ne_w2.T,
             ne_b2[None])
    for _ in range(6):
        g2 = gather_2np(table, cidx)
        table = _round_call(g2, init, table, *rargs,
                            interpret=tc_interpret)

    xg = gather_fin(table, uids_ext)
    return _head_call(
        xg, wt_ln_g[None], wt_ln_b[None], wt_w1.T, wt_b1[None], wt_w2.T,
        wt_b2.reshape(1, 1),
        vh_ln_g[None], vh_ln_b[None], vh_w1.T, vh_b1[None], vh_w2.T,
        vh_b2.reshape(1, 1),
        interpret=tc_interpret)


def kernel(raw_feats, uids, id_map, fe_bn_g, fe_bn_b, fe_w1, fe_b1, fe_w2,
           fe_b2, ne_ln_g, ne_ln_b, ne_w1, ne_b1, ne_w2, ne_b2, wt_ln_g,
           wt_ln_b, wt_w1, wt_b1, wt_w2, wt_b2, vh_ln_g, vh_ln_b, vh_w1,
           vh_b1, vh_w2, vh_b2):
    return _run(raw_feats, uids, id_map, fe_bn_g, fe_bn_b, fe_w1, fe_b1,
                fe_w2, fe_b2, ne_ln_g, ne_ln_b, ne_w1, ne_b1, ne_w2, ne_b2,
                wt_ln_g, wt_ln_b, wt_w1, wt_b1, wt_w2, wt_b2, vh_ln_g,
                vh_ln_b, vh_w1, vh_b1, vh_w2, vh_b2)
